# baseline (speedup 1.0000x reference)
import functools

import jax
import jax.numpy as jnp
from jax import lax
from jax.experimental import pallas as pl
from jax.experimental.pallas import tpu as pltpu
from jax.experimental.pallas import tpu_sc as plsc


def kernel(center_ids, context_ids, center_weight, center_biase, context_weight, context_biase):
    del context_ids
    B = center_ids.shape[0]
    D = center_weight.shape[1]
    L = 16

    info = plsc.get_sparse_core_info()
    NC, NS = info.num_cores, info.num_subcores
    NW = NC * NS
    n = B // NW
    C = 128
    n_chunks = n // C

    ids = center_ids.astype(jnp.int32)
    mesh = plsc.VectorSubcoreMesh(core_axis_name="c", subcore_axis_name="s")

    @functools.partial(
        pl.kernel,
        mesh=mesh,
        compiler_params=pltpu.CompilerParams(
            needs_layout_passes=False,
        ),
        out_type=jax.ShapeDtypeStruct((B,), jnp.float32),
        scratch_types=[
            pltpu.VMEM((n,), jnp.int32),
            pltpu.VMEM((C, D), jnp.float32),
            pltpu.VMEM((C, D), jnp.float32),
            pltpu.VMEM((C, 1), jnp.float32),
            pltpu.VMEM((C, 1), jnp.float32),
            pltpu.VMEM((n,), jnp.float32),
            pltpu.SemaphoreType.DMA,
        ],
    )
    def glove_sc(ids_hbm, cw_hbm, cb_hbm, xw_hbm, xb_hbm, out_hbm,
                 idx_v, cwb, xwb, cbb, xbb, out_v, sem):
        wid = lax.axis_index("s") * NC + lax.axis_index("c")
        base = wid * n

        pltpu.sync_copy(ids_hbm.at[pl.ds(base, n)], idx_v)
        lanes = lax.iota(jnp.int32, L)
        zeros = jnp.zeros((L,), jnp.int32)

        for c in range(n_chunks):
            def fire(g):
                idvec = idx_v[pl.ds(c * C + g * L, L)]
                for k in range(L):
                    row = idvec[k]
                    slot = g * L + k
                    pltpu.async_copy(cw_hbm.at[pl.ds(row, 1), pl.ds(0, D)],
                                     cwb.at[pl.ds(slot, 1), pl.ds(0, D)], sem)
                    pltpu.async_copy(xw_hbm.at[pl.ds(row, 1), pl.ds(0, D)],
                                     xwb.at[pl.ds(slot, 1), pl.ds(0, D)], sem)
                    pltpu.async_copy(cb_hbm.at[pl.ds(row, 1), pl.ds(0, 1)],
                                     cbb.at[pl.ds(slot, 1), pl.ds(0, 1)], sem)
                    pltpu.async_copy(xb_hbm.at[pl.ds(row, 1), pl.ds(0, 1)],
                                     xbb.at[pl.ds(slot, 1), pl.ds(0, 1)], sem)

            pl.loop(0, C // L)(fire)

            pltpu.make_async_copy(cw_hbm.at[pl.ds(0, C), pl.ds(0, D)], cwb, sem).wait()
            pltpu.make_async_copy(xw_hbm.at[pl.ds(0, C), pl.ds(0, D)], xwb, sem).wait()
            pltpu.make_async_copy(cb_hbm.at[pl.ds(0, C), pl.ds(0, 1)], cbb, sem).wait()
            pltpu.make_async_copy(xb_hbm.at[pl.ds(0, C), pl.ds(0, 1)], xbb, sem).wait()

            def comp(t):
                o = t * L
                rows = o + lanes
                acc = plsc.load_gather(cbb, [rows, zeros]) + plsc.load_gather(xbb, [rows, zeros])
                for k in range(L):
                    p = cwb[o + k, pl.ds(0, L)] * xwb[o + k, pl.ds(0, L)]
                    p = p + cwb[o + k, pl.ds(L, L)] * xwb[o + k, pl.ds(L, L)]
                    s = jnp.sum(p)
                    acc = acc + jnp.where(lanes == k, s, jnp.float32(0.0))
                out_v[pl.ds(c * C + o, L)] = acc

            pl.loop(0, C // L)(comp)

        pltpu.sync_copy(out_v, out_hbm.at[pl.ds(base, n)])

    out = glove_sc(ids, center_weight, center_biase, context_weight, context_biase)
    return out.reshape(B, 1)
